# Initial kernel scaffold; baseline (speedup 1.0000x reference)
import jax
import jax.numpy as jnp
from jax.experimental import pallas as pl


def kernel(x, edge_index, edge_attr, batch, W_in, b_in, We, be, W1, b1, W2, b2, eps, ln_g, ln_b, Wr1, br1, Wr2, br2, Wr3, br3):
    raise NotImplementedError("write your pallas kernel here")


# speedup vs baseline: 3.6840x; 3.6840x over previous
import functools

import jax
import jax.numpy as jnp
from jax import lax
from jax.experimental import pallas as pl
from jax.experimental.pallas import tpu as pltpu
from jax.experimental.pallas import tpu_sc as plsc

N_NODES = 10000
N_EDGES = 320000
D_FEAT = 128
D_EDGE = 16
HIDDEN = 128
N_LAYERS = 4
N_GRAPHS = 128

NB = 10
NODE_BLK = N_NODES // NB
EDGE_BLK = 8000

NWORK = 32
EDGES_PER_W = N_EDGES // NWORK
CHUNK = 80
NCHUNK = EDGES_PER_W // CHUNK
SUPER = 25
NSUPER = NCHUNK // SUPER
AGG_ROWS = 10240
ROWS_PER_TILE = AGG_ROWS // 16


def _proj_x_body(x_ref, w_ref, b_ref, o_ref):
    o_ref[...] = (
        jnp.dot(x_ref[...], w_ref[...], preferred_element_type=jnp.float32)
        + b_ref[...]
    )


def _proj_x(x, W_in, b_in):
    return pl.pallas_call(
        _proj_x_body,
        grid=(NB,),
        in_specs=[
            pl.BlockSpec((NODE_BLK, D_FEAT), lambda i: (i, 0)),
            pl.BlockSpec((D_FEAT, HIDDEN), lambda i: (0, 0)),
            pl.BlockSpec((1, HIDDEN), lambda i: (0, 0)),
        ],
        out_specs=pl.BlockSpec((NODE_BLK, HIDDEN), lambda i: (i, 0)),
        out_shape=jax.ShapeDtypeStruct((N_NODES, HIDDEN), jnp.float32),
    )(x, W_in, b_in.reshape(1, HIDDEN))


def _proj_e_body(a_ref, w_ref, b_ref, o_ref):
    o_ref[...] = (
        jnp.dot(a_ref[...], w_ref[...], preferred_element_type=jnp.float32)
        + b_ref[...]
    )


def _proj_e(edge_attr, We_l, be_l):
    neb = N_EDGES // EDGE_BLK
    return pl.pallas_call(
        _proj_e_body,
        grid=(neb,),
        in_specs=[
            pl.BlockSpec((EDGE_BLK, D_EDGE), lambda i: (i, 0)),
            pl.BlockSpec((D_EDGE, HIDDEN), lambda i: (0, 0)),
            pl.BlockSpec((1, HIDDEN), lambda i: (0, 0)),
        ],
        out_specs=pl.BlockSpec((EDGE_BLK, HIDDEN), lambda i: (i, 0)),
        out_shape=jax.ShapeDtypeStruct((N_EDGES, HIDDEN), jnp.float32),
    )(edge_attr, We_l, be_l.reshape(1, HIDDEN))


def _edge_sc_body(h_hbm, e_hbm, src_hbm, dst_hbm, out_hbm,
                  sidx, didx, hbuf, ebuf, agg, gsem):
    c = lax.axis_index("c")
    s = lax.axis_index("s")
    w = c * 16 + s

    zero = jnp.zeros((16,), jnp.float32)

    def zrow(i, carry):
        for k in range(HIDDEN // 16):
            hbuf[i, pl.ds(k * 16, 16)] = zero
        return carry

    lax.fori_loop(0, CHUNK, zrow, 0)
    r0 = pl.multiple_of(s * ROWS_PER_TILE, 8)
    for r in range(ROWS_PER_TILE // CHUNK):
        pltpu.sync_copy(hbuf, agg.at[pl.ds(r0 + r * CHUNK, CHUNK)])
    plsc.subcore_barrier()

    base = w * EDGES_PER_W

    def superblk(sb, carry):
        wsb = w * NSUPER + sb
        pltpu.sync_copy(src_hbm.at[wsb], sidx)
        pltpu.sync_copy(dst_hbm.at[wsb], didx)

        def chunk(j, carry1):
            row0 = pl.multiple_of(base + (sb * SUPER + j) * CHUNK, 8)
            cp = pltpu.async_copy(h_hbm.at[sidx.at[j]], hbuf, gsem)
            pltpu.sync_copy(e_hbm.at[pl.ds(row0, CHUNK)], ebuf)
            cp.wait()

            def rowfn(i, carry2):
                for k in range(HIDDEN // 16):
                    sl = pl.ds(k * 16, 16)
                    ebuf[i, sl] = jnp.maximum(hbuf[i, sl] + ebuf[i, sl], 0.0)
                return carry2

            lax.fori_loop(0, CHUNK, rowfn, 0)
            pltpu.sync_copy(ebuf, agg.at[didx.at[j]], add=True)
            return carry1

        lax.fori_loop(0, SUPER, chunk, 0)
        return carry

    lax.fori_loop(0, NSUPER, superblk, 0)
    plsc.subcore_barrier()

    pltpu.sync_copy(agg.at[pl.ds(r0, ROWS_PER_TILE)],
                    out_hbm.at[c, pl.ds(r0, ROWS_PER_TILE)])


def _edge_sc(h, e, src2, dst2):
    mesh = plsc.VectorSubcoreMesh(core_axis_name="c", subcore_axis_name="s")
    fn = pl.kernel(
        _edge_sc_body,
        out_type=jax.ShapeDtypeStruct((2, AGG_ROWS, HIDDEN), jnp.float32),
        mesh=mesh,
        scratch_types=[
            pltpu.VMEM((SUPER, CHUNK), jnp.int32),
            pltpu.VMEM((SUPER, CHUNK), jnp.int32),
            pltpu.VMEM((CHUNK, HIDDEN), jnp.float32),
            pltpu.VMEM((CHUNK, HIDDEN), jnp.float32),
            pltpu.VMEM_SHARED((AGG_ROWS, HIDDEN), jnp.float32),
            pltpu.SemaphoreType.DMA,
        ],
    )
    return fn(h, e, src2, dst2)


def _update_body(h_ref, a_ref, w1_ref, b1_ref, w2_ref, b2_ref,
                 al_ref, g_ref, be_ref, o_ref):
    h = h_ref[...]
    upd = al_ref[...] * h + a_ref[0] + a_ref[1]
    t = jnp.maximum(
        jnp.dot(upd, w1_ref[...], preferred_element_type=jnp.float32)
        + b1_ref[...], 0.0)
    z = jnp.dot(t, w2_ref[...], preferred_element_type=jnp.float32) + b2_ref[...]
    mu = jnp.mean(z, axis=-1, keepdims=True)
    zc = z - mu
    var = jnp.mean(zc * zc, axis=-1, keepdims=True)
    ln = zc * lax.rsqrt(var + 1e-5) * g_ref[...] + be_ref[...]
    o_ref[...] = h + ln


def _layer_update(h, agg2, W1_l, b1_l, W2_l, b2_l, alpha_l, g_l, be_l):
    vec = lambda v: v.reshape(1, HIDDEN)
    return pl.pallas_call(
        _update_body,
        grid=(NB,),
        in_specs=[
            pl.BlockSpec((NODE_BLK, HIDDEN), lambda i: (i, 0)),
            pl.BlockSpec((2, NODE_BLK, HIDDEN), lambda i: (0, i, 0)),
            pl.BlockSpec((HIDDEN, HIDDEN), lambda i: (0, 0)),
            pl.BlockSpec((1, HIDDEN), lambda i: (0, 0)),
            pl.BlockSpec((HIDDEN, HIDDEN), lambda i: (0, 0)),
            pl.BlockSpec((1, HIDDEN), lambda i: (0, 0)),
            pl.BlockSpec((1, HIDDEN), lambda i: (0, 0)),
            pl.BlockSpec((1, HIDDEN), lambda i: (0, 0)),
            pl.BlockSpec((1, HIDDEN), lambda i: (0, 0)),
        ],
        out_specs=pl.BlockSpec((NODE_BLK, HIDDEN), lambda i: (i, 0)),
        out_shape=jax.ShapeDtypeStruct((N_NODES, HIDDEN), jnp.float32),
    )(h, agg2, W1_l, vec(b1_l), W2_l, vec(b2_l), alpha_l, vec(g_l), vec(be_l))


def _pool_body(h_ref, b_ref, wr1_ref, br1_ref, wr2_ref, br2_ref,
               wr3_ref, br3_ref, o_ref, sums, cnts):
    i = pl.program_id(0)

    @pl.when(i == 0)
    def _():
        sums[...] = jnp.zeros_like(sums)
        cnts[...] = jnp.zeros_like(cnts)

    bids = b_ref[0, 0, :].reshape(NODE_BLK, 1)
    gids = lax.broadcasted_iota(jnp.int32, (NODE_BLK, N_GRAPHS), 1)
    onehot = (bids == gids).astype(jnp.float32)
    dn = (((0,), (0,)), ((), ()))
    sums[...] += lax.dot_general(onehot, h_ref[...], dn,
                                 preferred_element_type=jnp.float32)
    cnts[...] += lax.dot_general(onehot, jnp.ones((NODE_BLK, HIDDEN), jnp.float32),
                                 dn, preferred_element_type=jnp.float32)

    @pl.when(i == NB - 1)
    def _():
        g = sums[...] / jnp.maximum(cnts[...], 1.0)
        a = jnp.maximum(
            jnp.dot(g, wr1_ref[...], preferred_element_type=jnp.float32)
            + br1_ref[...], 0.0)
        a = jnp.maximum(
            jnp.dot(a, wr2_ref[...], preferred_element_type=jnp.float32)
            + br2_ref[...], 0.0)
        o_ref[...] = (
            jnp.dot(a, wr3_ref[...], preferred_element_type=jnp.float32)
            + br3_ref[...])


def _pool_head(h, batch3, Wr1, br1, Wr2, br2, Wr3, br3):
    return pl.pallas_call(
        _pool_body,
        grid=(NB,),
        in_specs=[
            pl.BlockSpec((NODE_BLK, HIDDEN), lambda i: (i, 0)),
            pl.BlockSpec((1, 1, NODE_BLK), lambda i: (i, 0, 0)),
            pl.BlockSpec((HIDDEN, 128), lambda i: (0, 0)),
            pl.BlockSpec((1, 128), lambda i: (0, 0)),
            pl.BlockSpec((128, 64), lambda i: (0, 0)),
            pl.BlockSpec((1, 64), lambda i: (0, 0)),
            pl.BlockSpec((64, 1), lambda i: (0, 0)),
            pl.BlockSpec((1, 1), lambda i: (0, 0)),
        ],
        out_specs=pl.BlockSpec((N_GRAPHS, 1), lambda i: (0, 0)),
        out_shape=jax.ShapeDtypeStruct((N_GRAPHS, 1), jnp.float32),
        scratch_shapes=[
            pltpu.VMEM((N_GRAPHS, HIDDEN), jnp.float32),
            pltpu.VMEM((N_GRAPHS, HIDDEN), jnp.float32),
        ],
    )(h, batch3, Wr1, br1.reshape(1, 128), Wr2, br2.reshape(1, 64),
      Wr3, br3.reshape(1, 1))


def kernel(x, edge_index, edge_attr, batch, W_in, b_in, We, be, W1, b1, W2, b2,
           eps, ln_g, ln_b, Wr1, br1, Wr2, br2, Wr3, br3):
    src2 = edge_index[0].astype(jnp.int32).reshape(NWORK * NSUPER, SUPER, CHUNK)
    dst2 = edge_index[1].astype(jnp.int32).reshape(NWORK * NSUPER, SUPER, CHUNK)
    batch3 = batch.astype(jnp.int32).reshape(NB, 1, NODE_BLK)

    h = _proj_x(x, W_in, b_in)
    for l in range(N_LAYERS):
        e = _proj_e(edge_attr, We[l], be[l])
        agg2 = _edge_sc(h, e, src2, dst2)
        alpha = jnp.full((1, HIDDEN), 1.0 + eps[l], jnp.float32)
        h = _layer_update(h, agg2, W1[l], b1[l], W2[l], b2[l],
                          alpha, ln_g[l], ln_b[l])
    return _pool_head(h, batch3, Wr1, br1, Wr2, br2, Wr3, br3)


# speedup vs baseline: 4.6301x; 1.2568x over previous
import functools

import jax
import jax.numpy as jnp
from jax import lax
from jax.experimental import pallas as pl
from jax.experimental.pallas import tpu as pltpu
from jax.experimental.pallas import tpu_sc as plsc

N_NODES = 10000
N_EDGES = 320000
D_FEAT = 128
D_EDGE = 16
HIDDEN = 128
N_LAYERS = 4
N_GRAPHS = 128

NB = 10
NODE_BLK = N_NODES // NB
EDGE_BLK = 8000

NWORK = 32
EDGES_PER_W = N_EDGES // NWORK
CHUNK = 40
NCHUNK = EDGES_PER_W // CHUNK
SUPER = 50
NSUPER = NCHUNK // SUPER
AGG_ROWS = 10240
ROWS_PER_TILE = AGG_ROWS // 16


def _proj_x_body(x_ref, w_ref, b_ref, o_ref):
    o_ref[...] = (
        jnp.dot(x_ref[...], w_ref[...], preferred_element_type=jnp.float32)
        + b_ref[...]
    )


def _proj_x(x, W_in, b_in):
    return pl.pallas_call(
        _proj_x_body,
        grid=(NB,),
        in_specs=[
            pl.BlockSpec((NODE_BLK, D_FEAT), lambda i: (i, 0)),
            pl.BlockSpec((D_FEAT, HIDDEN), lambda i: (0, 0)),
            pl.BlockSpec((1, HIDDEN), lambda i: (0, 0)),
        ],
        out_specs=pl.BlockSpec((NODE_BLK, HIDDEN), lambda i: (i, 0)),
        out_shape=jax.ShapeDtypeStruct((N_NODES, HIDDEN), jnp.float32),
    )(x, W_in, b_in.reshape(1, HIDDEN))


def _proj_e_body(a_ref, w_ref, b_ref, o_ref):
    o_ref[...] = (
        jnp.dot(a_ref[...], w_ref[...], preferred_element_type=jnp.float32)
        + b_ref[...]
    )


def _proj_e(edge_attr, We_l, be_l):
    neb = N_EDGES // EDGE_BLK
    return pl.pallas_call(
        _proj_e_body,
        grid=(neb,),
        in_specs=[
            pl.BlockSpec((EDGE_BLK, D_EDGE), lambda i: (i, 0)),
            pl.BlockSpec((D_EDGE, HIDDEN), lambda i: (0, 0)),
            pl.BlockSpec((1, HIDDEN), lambda i: (0, 0)),
        ],
        out_specs=pl.BlockSpec((EDGE_BLK, HIDDEN), lambda i: (i, 0)),
        out_shape=jax.ShapeDtypeStruct((N_EDGES, HIDDEN), jnp.float32),
    )(edge_attr, We_l, be_l.reshape(1, HIDDEN))


def _edge_sc_body(h_hbm, e_hbm, src_hbm, dst_hbm, out_hbm,
                  sidx, didx, hbuf, ebuf, agg,
                  gsem0, gsem1, esem0, esem1):
    c = lax.axis_index("c")
    s = lax.axis_index("s")
    w = c * 16 + s
    gsems = (gsem0, gsem1)
    esems = (esem0, esem1)

    zero = jnp.zeros((16,), jnp.float32)

    def zrow(i, carry):
        for k in range(HIDDEN // 16):
            hbuf[0, i, pl.ds(k * 16, 16)] = zero
        return carry

    lax.fori_loop(0, CHUNK, zrow, 0)
    r0 = pl.multiple_of(s * ROWS_PER_TILE, 8)
    for r in range(ROWS_PER_TILE // CHUNK):
        pltpu.sync_copy(hbuf.at[0], agg.at[pl.ds(r0 + r * CHUNK, CHUNK)])
    plsc.subcore_barrier()

    base = w * EDGES_PER_W

    def start_dmas(sb, j, b):
        row0 = pl.multiple_of(base + (sb * SUPER + j) * CHUNK, 8)
        pltpu.async_copy(h_hbm.at[sidx.at[j]], hbuf.at[b], gsems[b])
        pltpu.async_copy(e_hbm.at[pl.ds(row0, CHUNK)], ebuf.at[b], esems[b])

    def wait_dmas(sb, j, b):
        row0 = pl.multiple_of(base + (sb * SUPER + j) * CHUNK, 8)
        pltpu.make_async_copy(h_hbm.at[sidx.at[j]], hbuf.at[b], gsems[b]).wait()
        pltpu.make_async_copy(e_hbm.at[pl.ds(row0, CHUNK)], ebuf.at[b],
                              esems[b]).wait()

    def superblk(sb, carry):
        wsb = w * NSUPER + sb
        pltpu.sync_copy(src_hbm.at[wsb], sidx)
        pltpu.sync_copy(dst_hbm.at[wsb], didx)

        for b in range(2):
            start_dmas(sb, b, b)

        def pair(p, carry1):
            for b in range(2):
                j = p * 2 + b
                wait_dmas(sb, j, b)

                def rowfn(i, carry2):
                    for k in range(HIDDEN // 16):
                        sl = pl.ds(k * 16, 16)
                        ebuf[b, i, sl] = jnp.maximum(
                            hbuf[b, i, sl] + ebuf[b, i, sl], 0.0)
                    return carry2

                lax.fori_loop(0, CHUNK, rowfn, 0)
                pltpu.sync_copy(ebuf.at[b], agg.at[didx.at[j]], add=True)

                @pl.when(j + 2 < SUPER)
                def _():
                    start_dmas(sb, j + 2, b)
            return carry1

        lax.fori_loop(0, SUPER // 2, pair, 0)
        return carry

    lax.fori_loop(0, NSUPER, superblk, 0)
    plsc.subcore_barrier()

    pltpu.sync_copy(agg.at[pl.ds(r0, ROWS_PER_TILE)],
                    out_hbm.at[c, pl.ds(r0, ROWS_PER_TILE)])


def _edge_sc(h, e, src2, dst2):
    mesh = plsc.VectorSubcoreMesh(core_axis_name="c", subcore_axis_name="s")
    fn = pl.kernel(
        _edge_sc_body,
        out_type=jax.ShapeDtypeStruct((2, AGG_ROWS, HIDDEN), jnp.float32),
        mesh=mesh,
        scratch_types=[
            pltpu.VMEM((SUPER, CHUNK), jnp.int32),
            pltpu.VMEM((SUPER, CHUNK), jnp.int32),
            pltpu.VMEM((2, CHUNK, HIDDEN), jnp.float32),
            pltpu.VMEM((2, CHUNK, HIDDEN), jnp.float32),
            pltpu.VMEM_SHARED((AGG_ROWS, HIDDEN), jnp.float32),
            pltpu.SemaphoreType.DMA,
            pltpu.SemaphoreType.DMA,
            pltpu.SemaphoreType.DMA,
            pltpu.SemaphoreType.DMA,
        ],
    )
    return fn(h, e, src2, dst2)


def _update_body(h_ref, a_ref, w1_ref, b1_ref, w2_ref, b2_ref,
                 al_ref, g_ref, be_ref, o_ref):
    h = h_ref[...]
    upd = al_ref[...] * h + a_ref[0] + a_ref[1]
    t = jnp.maximum(
        jnp.dot(upd, w1_ref[...], preferred_element_type=jnp.float32)
        + b1_ref[...], 0.0)
    z = jnp.dot(t, w2_ref[...], preferred_element_type=jnp.float32) + b2_ref[...]
    mu = jnp.mean(z, axis=-1, keepdims=True)
    zc = z - mu
    var = jnp.mean(zc * zc, axis=-1, keepdims=True)
    ln = zc * lax.rsqrt(var + 1e-5) * g_ref[...] + be_ref[...]
    o_ref[...] = h + ln


def _layer_update(h, agg2, W1_l, b1_l, W2_l, b2_l, alpha_l, g_l, be_l):
    vec = lambda v: v.reshape(1, HIDDEN)
    return pl.pallas_call(
        _update_body,
        grid=(NB,),
        in_specs=[
            pl.BlockSpec((NODE_BLK, HIDDEN), lambda i: (i, 0)),
            pl.BlockSpec((2, NODE_BLK, HIDDEN), lambda i: (0, i, 0)),
            pl.BlockSpec((HIDDEN, HIDDEN), lambda i: (0, 0)),
            pl.BlockSpec((1, HIDDEN), lambda i: (0, 0)),
            pl.BlockSpec((HIDDEN, HIDDEN), lambda i: (0, 0)),
            pl.BlockSpec((1, HIDDEN), lambda i: (0, 0)),
            pl.BlockSpec((1, HIDDEN), lambda i: (0, 0)),
            pl.BlockSpec((1, HIDDEN), lambda i: (0, 0)),
            pl.BlockSpec((1, HIDDEN), lambda i: (0, 0)),
        ],
        out_specs=pl.BlockSpec((NODE_BLK, HIDDEN), lambda i: (i, 0)),
        out_shape=jax.ShapeDtypeStruct((N_NODES, HIDDEN), jnp.float32),
    )(h, agg2, W1_l, vec(b1_l), W2_l, vec(b2_l), alpha_l, vec(g_l), vec(be_l))


def _pool_body(h_ref, b_ref, wr1_ref, br1_ref, wr2_ref, br2_ref,
               wr3_ref, br3_ref, o_ref, sums, cnts):
    i = pl.program_id(0)

    @pl.when(i == 0)
    def _():
        sums[...] = jnp.zeros_like(sums)
        cnts[...] = jnp.zeros_like(cnts)

    bids = b_ref[0, 0, :].reshape(NODE_BLK, 1)
    gids = lax.broadcasted_iota(jnp.int32, (NODE_BLK, N_GRAPHS), 1)
    onehot = (bids == gids).astype(jnp.float32)
    dn = (((0,), (0,)), ((), ()))
    sums[...] += lax.dot_general(onehot, h_ref[...], dn,
                                 preferred_element_type=jnp.float32)
    cnts[...] += lax.dot_general(onehot, jnp.ones((NODE_BLK, HIDDEN), jnp.float32),
                                 dn, preferred_element_type=jnp.float32)

    @pl.when(i == NB - 1)
    def _():
        g = sums[...] / jnp.maximum(cnts[...], 1.0)
        a = jnp.maximum(
            jnp.dot(g, wr1_ref[...], preferred_element_type=jnp.float32)
            + br1_ref[...], 0.0)
        a = jnp.maximum(
            jnp.dot(a, wr2_ref[...], preferred_element_type=jnp.float32)
            + br2_ref[...], 0.0)
        o_ref[...] = (
            jnp.dot(a, wr3_ref[...], preferred_element_type=jnp.float32)
            + br3_ref[...])


def _pool_head(h, batch3, Wr1, br1, Wr2, br2, Wr3, br3):
    return pl.pallas_call(
        _pool_body,
        grid=(NB,),
        in_specs=[
            pl.BlockSpec((NODE_BLK, HIDDEN), lambda i: (i, 0)),
            pl.BlockSpec((1, 1, NODE_BLK), lambda i: (i, 0, 0)),
            pl.BlockSpec((HIDDEN, 128), lambda i: (0, 0)),
            pl.BlockSpec((1, 128), lambda i: (0, 0)),
            pl.BlockSpec((128, 64), lambda i: (0, 0)),
            pl.BlockSpec((1, 64), lambda i: (0, 0)),
            pl.BlockSpec((64, 1), lambda i: (0, 0)),
            pl.BlockSpec((1, 1), lambda i: (0, 0)),
        ],
        out_specs=pl.BlockSpec((N_GRAPHS, 1), lambda i: (0, 0)),
        out_shape=jax.ShapeDtypeStruct((N_GRAPHS, 1), jnp.float32),
        scratch_shapes=[
            pltpu.VMEM((N_GRAPHS, HIDDEN), jnp.float32),
            pltpu.VMEM((N_GRAPHS, HIDDEN), jnp.float32),
        ],
    )(h, batch3, Wr1, br1.reshape(1, 128), Wr2, br2.reshape(1, 64),
      Wr3, br3.reshape(1, 1))


def kernel(x, edge_index, edge_attr, batch, W_in, b_in, We, be, W1, b1, W2, b2,
           eps, ln_g, ln_b, Wr1, br1, Wr2, br2, Wr3, br3):
    src2 = edge_index[0].astype(jnp.int32).reshape(NWORK * NSUPER, SUPER, CHUNK)
    dst2 = edge_index[1].astype(jnp.int32).reshape(NWORK * NSUPER, SUPER, CHUNK)
    batch3 = batch.astype(jnp.int32).reshape(NB, 1, NODE_BLK)

    h = _proj_x(x, W_in, b_in)
    for l in range(N_LAYERS):
        e = _proj_e(edge_attr, We[l], be[l])
        agg2 = _edge_sc(h, e, src2, dst2)
        alpha = jnp.full((1, HIDDEN), 1.0 + eps[l], jnp.float32)
        h = _layer_update(h, agg2, W1[l], b1[l], W2[l], b2[l],
                          alpha, ln_g[l], ln_b[l])
    return _pool_head(h, batch3, Wr1, br1, Wr2, br2, Wr3, br3)
